# Initial kernel scaffold; baseline (speedup 1.0000x reference)
import jax
import jax.numpy as jnp
from jax.experimental import pallas as pl


def kernel(x, edge_index, batch, prior_logits, emission_logits, W, b):
    raise NotImplementedError("write your pallas kernel here")


# speedup vs baseline: 51.3793x; 51.3793x over previous
import functools

import jax
import jax.numpy as jnp
import numpy as np
from jax import lax
from jax.experimental import pallas as pl
from jax.experimental.pallas import tpu as pltpu
from jax.experimental.pallas import tpu_sc as plsc

N_GRAPHS = 256
M_VALS = 32
N_GEN = 16
C_MIX = 8
CU = N_GEN * (N_GEN - 1) // 2

_NC = 2
_NS = 16
_NW = _NC * _NS
_L = 16

_NBINS = (N_GRAPHS + 1) * M_VALS


def _cm_np():
    cm = np.zeros((N_GEN, CU), dtype=np.float32)
    p, s = 0, 1
    for i in range(CU):
        cm[p, i] = 1.0
        cm[s, i] = -1.0
        if s == N_GEN - 1:
            p = p + 1
            s = p
        s = s + 1
    return cm


_CM = _cm_np()


def _make_hist_kernel(n_pad):
    chunk = n_pad // _NW
    n_vec = chunk // _L
    mesh = plsc.VectorSubcoreMesh(core_axis_name="c", subcore_axis_name="s")

    @functools.partial(
        pl.kernel,
        out_type=jax.ShapeDtypeStruct((_NW, _NBINS), jnp.int32),
        mesh=mesh,
        scratch_types=[
            pltpu.VMEM((chunk,), jnp.int32),
            pltpu.VMEM((chunk,), jnp.int32),
            pltpu.VMEM((_NBINS,), jnp.int32),
        ],
        compiler_params=pltpu.CompilerParams(needs_layout_passes=False),
    )
    def hist_kernel(x_hbm, b_hbm, out_hbm, x_v, b_v, hist_v):
        wid = lax.axis_index("s") * _NC + lax.axis_index("c")
        base = wid * chunk
        pltpu.sync_copy(x_hbm.at[pl.ds(base, chunk)], x_v)
        pltpu.sync_copy(b_hbm.at[pl.ds(base, chunk)], b_v)

        zeros = jnp.zeros((_L,), jnp.int32)

        def zero_body(i, _):
            hist_v[pl.ds(i * _L, _L)] = zeros
            return 0

        lax.fori_loop(0, _NBINS // _L, zero_body, 0)

        ones = jnp.ones((_L,), jnp.int32)

        def body(j, _):
            xk = x_v[pl.ds(j * _L, _L)]
            bk = b_v[pl.ds(j * _L, _L)]
            key = bk * M_VALS + xk
            plsc.addupdate_scatter(hist_v, [key], ones)
            return 0

        lax.fori_loop(0, n_vec, body, 0)

        pltpu.sync_copy(hist_v, out_hbm.at[wid])

    return hist_kernel


def _dense_body(counts_ref, lp_ref, le_ref, cm_ref, w_ref, b_ref, out_ref):
    lp_raw = lp_ref[...]
    lp = lp_raw - (
        jnp.max(lp_raw, axis=-1, keepdims=True)
        + jnp.log(
            jnp.sum(
                jnp.exp(lp_raw - jnp.max(lp_raw, axis=-1, keepdims=True)),
                axis=-1,
                keepdims=True,
            )
        )
    )

    le_raw = le_ref[...]
    le_max = jnp.max(le_raw, axis=-1, keepdims=True)
    le_lse = le_max + jnp.log(
        jnp.sum(jnp.exp(le_raw - le_max), axis=-1, keepdims=True)
    )
    terms = []
    for c in range(C_MIX):
        t = lp[:, c][:, None] + (le_raw[:, c, :] - le_lse[:, c, :])
        terms.append(t)
    mx = terms[0]
    for c in range(1, C_MIX):
        mx = jnp.maximum(mx, terms[c])
    ssum = jnp.exp(terms[0] - mx)
    for c in range(1, C_MIX):
        ssum = ssum + jnp.exp(terms[c] - mx)
    ll = mx + jnp.log(ssum)

    cnt = counts_ref[0]
    for w in range(1, _NW):
        cnt = cnt + counts_ref[w]
    cnt = cnt.astype(jnp.float32)

    pooled = jnp.sum(cnt[:, None, :] * ll[None, :, :], axis=-1)

    cu = jnp.tanh(
        jax.lax.dot(pooled, cm_ref[...], preferred_element_type=jnp.float32)
    )
    out = jax.lax.dot(cu, w_ref[...], preferred_element_type=jnp.float32)
    out_ref[...] = out + b_ref[...]


def kernel(x, edge_index, batch, prior_logits, emission_logits, W, b):
    del edge_index
    n = x.shape[0]
    chunk = ((n + _NW * _L - 1) // (_NW * _L)) * _L
    n_pad = chunk * _NW
    pad = n_pad - n

    x = x.astype(jnp.int32)
    batch = batch.astype(jnp.int32)
    if pad:
        x = jnp.concatenate([x, jnp.zeros((pad,), jnp.int32)])
        batch = jnp.concatenate(
            [batch, jnp.full((pad,), N_GRAPHS, jnp.int32)]
        )

    counts = _make_hist_kernel(n_pad)(x, batch)
    counts3 = counts[:, : N_GRAPHS * M_VALS].reshape(_NW, N_GRAPHS, M_VALS)

    out = pl.pallas_call(
        _dense_body,
        out_shape=jax.ShapeDtypeStruct((N_GRAPHS, 10), jnp.float32),
    )(
        counts3,
        prior_logits,
        emission_logits,
        jnp.asarray(_CM),
        W,
        b.reshape(1, 10),
    )
    return out


# speedup vs baseline: 59.5139x; 1.1583x over previous
import functools

import jax
import jax.numpy as jnp
import numpy as np
from jax import lax
from jax.experimental import pallas as pl
from jax.experimental.pallas import tpu as pltpu
from jax.experimental.pallas import tpu_sc as plsc

N_GRAPHS = 256
M_VALS = 32
N_GEN = 16
C_MIX = 8
CU = N_GEN * (N_GEN - 1) // 2

_NC = 2
_NS = 16
_NW = _NC * _NS
_L = 16

_HROWS = N_GRAPHS + 1


def _cm_np():
    cm = np.zeros((N_GEN, CU), dtype=np.float32)
    p, s = 0, 1
    for i in range(CU):
        cm[p, i] = 1.0
        cm[s, i] = -1.0
        if s == N_GEN - 1:
            p = p + 1
            s = p
        s = s + 1
    return cm


_CM = _cm_np()


def _make_hist_kernel(n16):
    q = n16 // _L
    v_lo = q // _NW
    rem = q % _NW
    hi_sz = (v_lo + 1) * _L
    lo_sz = v_lo * _L
    mesh = plsc.VectorSubcoreMesh(core_axis_name="c", subcore_axis_name="s")

    @functools.partial(
        pl.kernel,
        out_type=jax.ShapeDtypeStruct((_NW, _HROWS, M_VALS), jnp.int32),
        mesh=mesh,
        scratch_types=[
            pltpu.VMEM((max(hi_sz, _L),), jnp.int32),
            pltpu.VMEM((max(hi_sz, _L),), jnp.int32),
            pltpu.VMEM((_HROWS, M_VALS), jnp.int32),
        ],
        compiler_params=pltpu.CompilerParams(needs_layout_passes=False),
    )
    def hist_kernel(x_hbm, b_hbm, out_hbm, x_v, b_v, hist_v):
        wid = lax.axis_index("s") * _NC + lax.axis_index("c")

        zeros = jnp.zeros((_L,), jnp.int32)

        def zero_body(i, _):
            hist_v[i, pl.ds(0, _L)] = zeros
            hist_v[i, pl.ds(_L, _L)] = zeros
            return 0

        lax.fori_loop(0, _HROWS, zero_body, 0)

        ones = jnp.ones((_L,), jnp.int32)

        def accumulate(n_vec):
            def body(j, _):
                xk = x_v[pl.ds(j * _L, _L)]
                bk = b_v[pl.ds(j * _L, _L)]
                plsc.addupdate_scatter(hist_v, [bk, xk], ones)
                return 0

            lax.fori_loop(0, n_vec, body, 0)

        if rem:
            @pl.when(wid < rem)
            def _():
                base = wid * hi_sz
                pltpu.sync_copy(x_hbm.at[pl.ds(base, hi_sz)],
                                x_v.at[pl.ds(0, hi_sz)])
                pltpu.sync_copy(b_hbm.at[pl.ds(base, hi_sz)],
                                b_v.at[pl.ds(0, hi_sz)])
                accumulate(v_lo + 1)

        if lo_sz:
            @pl.when(wid >= rem)
            def _():
                base = rem * hi_sz + (wid - rem) * lo_sz
                pltpu.sync_copy(x_hbm.at[pl.ds(base, lo_sz)],
                                x_v.at[pl.ds(0, lo_sz)])
                pltpu.sync_copy(b_hbm.at[pl.ds(base, lo_sz)],
                                b_v.at[pl.ds(0, lo_sz)])
                accumulate(v_lo)

        pltpu.sync_copy(hist_v, out_hbm.at[wid])

    return hist_kernel


def _dense_body(counts_ref, lp_ref, le_ref, cm_ref, w_ref, b_ref, out_ref):
    lp_raw = lp_ref[...]
    lp_max = jnp.max(lp_raw, axis=-1, keepdims=True)
    lp = lp_raw - (
        lp_max
        + jnp.log(jnp.sum(jnp.exp(lp_raw - lp_max), axis=-1, keepdims=True))
    )

    le_raw = le_ref[...]
    le_max = jnp.max(le_raw, axis=-1, keepdims=True)
    le_lse = le_max + jnp.log(
        jnp.sum(jnp.exp(le_raw - le_max), axis=-1, keepdims=True)
    )
    terms = []
    for c in range(C_MIX):
        t = lp[:, c][:, None] + (le_raw[:, c, :] - le_lse[:, c, :])
        terms.append(t)
    mx = terms[0]
    for c in range(1, C_MIX):
        mx = jnp.maximum(mx, terms[c])
    ssum = jnp.exp(terms[0] - mx)
    for c in range(1, C_MIX):
        ssum = ssum + jnp.exp(terms[c] - mx)
    ll = mx + jnp.log(ssum)

    cnt = counts_ref[0, :N_GRAPHS, :]
    for w in range(1, _NW):
        cnt = cnt + counts_ref[w, :N_GRAPHS, :]
    cnt = cnt.astype(jnp.float32)

    pooled = jnp.sum(cnt[:, None, :] * ll[None, :, :], axis=-1)

    cu = jnp.tanh(
        jax.lax.dot(pooled, cm_ref[...], preferred_element_type=jnp.float32)
    )
    out = jax.lax.dot(cu, w_ref[...], preferred_element_type=jnp.float32)
    out_ref[...] = out + b_ref[...]


def kernel(x, edge_index, batch, prior_logits, emission_logits, W, b):
    del edge_index
    n = x.shape[0]
    x = x.astype(jnp.int32)
    batch = batch.astype(jnp.int32)

    n16 = ((n + _L - 1) // _L) * _L
    if n16 != n:
        pad = n16 - n
        x = jnp.concatenate([x, jnp.zeros((pad,), jnp.int32)])
        batch = jnp.concatenate([batch, jnp.full((pad,), N_GRAPHS, jnp.int32)])

    counts = _make_hist_kernel(n16)(x, batch)

    out = pl.pallas_call(
        _dense_body,
        out_shape=jax.ShapeDtypeStruct((N_GRAPHS, 10), jnp.float32),
    )(
        counts,
        prior_logits,
        emission_logits,
        jnp.asarray(_CM),
        W,
        b.reshape(1, 10),
    )
    return out
